# Initial kernel scaffold; baseline (speedup 1.0000x reference)
import jax
import jax.numpy as jnp
from jax.experimental import pallas as pl


def kernel(x, time_day):
    raise NotImplementedError("write your pallas kernel here")


# speedup vs baseline: 2.9438x; 2.9438x over previous
import jax
import jax.numpy as jnp
from jax import lax
from jax.experimental import pallas as pl
from jax.experimental.pallas import tpu as pltpu
from jax.experimental.pallas import tpu_sc as plsc

B = 64
N = 10000
F = 64
T = 288

NC, NS, L = 2, 16, 16
NW = NC * NS
BPW = B // NW
NB = 400
CH = N // NB
GPC = NB // L


def _body(xs_hbm, tab_hbm, out_hbm, tab_v, xs_v, ob_v):
    wid = lax.axis_index("s") * NC + lax.axis_index("c")
    pltpu.sync_copy(tab_hbm, tab_v)

    def per_batch(i, carry):
        b = wid * BPW + i
        pltpu.sync_copy(xs_hbm.at[b], xs_v)

        def per_chunk(c, carry):
            def per_group(g, carry):
                xv = xs_v[pl.ds(c * NB + g * L, L)]
                t = jnp.clip((xv * 288.0).astype(jnp.int32), 0, T - 1)
                for f in range(F):
                    vals = plsc.load_gather(tab_v, [t + f * T])
                    ob_v[f, pl.ds(g * L, L)] = vals
                return carry

            lax.fori_loop(0, GPC, per_group, 0)
            pltpu.sync_copy(ob_v, out_hbm.at[b, :, pl.ds(c * NB, NB)])
            return carry

        lax.fori_loop(0, CH, per_chunk, 0)
        return carry

    lax.fori_loop(0, BPW, per_batch, 0)


_sc = pl.kernel(
    _body,
    out_type=jax.ShapeDtypeStruct((B, F, N), jnp.float32),
    mesh=plsc.VectorSubcoreMesh(
        core_axis_name="c", subcore_axis_name="s",
        num_cores=NC, num_subcores=NS,
    ),
    scratch_types=[
        pltpu.VMEM((F * T,), jnp.float32),
        pltpu.VMEM((N,), jnp.float32),
        pltpu.VMEM((F, NB), jnp.float32),
    ],
    compiler_params=pltpu.CompilerParams(
        use_tc_tiling_on_sc=False, needs_layout_passes=False,
    ),
)


def kernel(x, time_day):
    xs = x[:, -1, :, 1]
    tab_t = time_day.T.reshape(F * T)
    return _sc(xs, tab_t)[..., None]


# speedup vs baseline: 3.7907x; 1.2877x over previous
import jax
import jax.numpy as jnp
from jax import lax
from jax.experimental import pallas as pl
from jax.experimental.pallas import tpu as pltpu
from jax.experimental.pallas import tpu_sc as plsc

B = 64
N = 10000
F = 64
T = 288

NC, NS, L = 2, 16, 16
NW = NC * NS
BPW = B // NW
NB = 400
CH = N // NB
GPC = NB // L


def _body(xs_hbm, tab_hbm, out_hbm, tab_v, xs_v, ob_v):
    wid = lax.axis_index("s") * NC + lax.axis_index("c")
    pltpu.sync_copy(tab_hbm, tab_v)

    def per_batch(i, carry):
        b = wid * BPW + i
        pltpu.sync_copy(xs_hbm.at[b], xs_v)

        def per_chunk(c, carry):
            @plsc.parallel_loop(0, GPC, unroll=2)
            def per_group(g):
                xv = xs_v[pl.ds(c * NB + g * L, L)]
                t = jnp.clip((xv * 288.0).astype(jnp.int32), 0, T - 1)
                for f in range(F):
                    vals = plsc.load_gather(tab_v, [t + f * T])
                    ob_v[f, pl.ds(g * L, L)] = vals
            pltpu.sync_copy(ob_v, out_hbm.at[b, :, pl.ds(c * NB, NB)])
            return carry

        lax.fori_loop(0, CH, per_chunk, 0)
        return carry

    lax.fori_loop(0, BPW, per_batch, 0)


_sc = pl.kernel(
    _body,
    out_type=jax.ShapeDtypeStruct((B, F, N), jnp.float32),
    mesh=plsc.VectorSubcoreMesh(
        core_axis_name="c", subcore_axis_name="s",
        num_cores=NC, num_subcores=NS,
    ),
    scratch_types=[
        pltpu.VMEM((F * T,), jnp.float32),
        pltpu.VMEM((N,), jnp.float32),
        pltpu.VMEM((F, NB), jnp.float32),
    ],
    compiler_params=pltpu.CompilerParams(
        use_tc_tiling_on_sc=False, needs_layout_passes=False,
        disable_bounds_checks=True,
    ),
)


def kernel(x, time_day):
    xs = x[:, -1, :, 1]
    tab_t = time_day.T.reshape(F * T)
    return _sc(xs, tab_t)[..., None]


# speedup vs baseline: 4.1065x; 1.0833x over previous
import jax
import jax.numpy as jnp
from jax import lax
from jax.experimental import pallas as pl
from jax.experimental.pallas import tpu as pltpu
from jax.experimental.pallas import tpu_sc as plsc

B = 64
N = 10000
F = 64
T = 288

NC, NS, L = 2, 16, 16
NW = NC * NS
BPW = B // NW
NB = 400
CH = N // NB
GPC = NB // L
TPW = BPW * CH


def _body(xs_hbm, tab_hbm, out_hbm, tab_v, xs_v, ob, sems):
    wid = lax.axis_index("s") * NC + lax.axis_index("c")
    b0 = wid * BPW
    pltpu.sync_copy(tab_hbm, tab_v)
    for i in range(BPW):
        pltpu.sync_copy(xs_hbm.at[b0 + i], xs_v.at[i])

    def pair(kk, carry):
        for j in range(2):
            t = kk * 2 + j
            bl = t // CH
            c = t % CH
            dst = out_hbm.at[b0 + bl, :, pl.ds(c * NB, NB)]

            @pl.when(kk > 0)
            def _wait_prev():
                pltpu.make_async_copy(ob[j], dst, sems[j]).wait()

            @plsc.parallel_loop(0, GPC, unroll=5)
            def per_group(g):
                xv = xs_v[bl, pl.ds(c * NB + g * L, L)]
                tt = jnp.clip((xv * 288.0).astype(jnp.int32), 0, T - 1)
                for f in range(F):
                    vals = plsc.load_gather(tab_v, [tt + f * T])
                    ob[j][f, pl.ds(g * L, L)] = vals

            pltpu.async_copy(ob[j], dst, sems[j])
        return carry

    lax.fori_loop(0, TPW // 2, pair, 0)
    for j in range(2):
        dst = out_hbm.at[b0, :, pl.ds(0, NB)]
        pltpu.make_async_copy(ob[j], dst, sems[j]).wait()


_sc = pl.kernel(
    _body,
    out_type=jax.ShapeDtypeStruct((B, F, N), jnp.float32),
    mesh=plsc.VectorSubcoreMesh(
        core_axis_name="c", subcore_axis_name="s",
        num_cores=NC, num_subcores=NS,
    ),
    scratch_types=[
        pltpu.VMEM((F * T,), jnp.float32),
        pltpu.VMEM((BPW, N), jnp.float32),
        [pltpu.VMEM((F, NB), jnp.float32)] * 2,
        [pltpu.SemaphoreType.DMA] * 2,
    ],
    compiler_params=pltpu.CompilerParams(
        use_tc_tiling_on_sc=False, needs_layout_passes=False,
        disable_bounds_checks=True,
    ),
)


def kernel(x, time_day):
    xs = x[:, -1, :, 1]
    tab_t = time_day.T.reshape(F * T)
    return _sc(xs, tab_t)[..., None]
